# baseline (speedup 1.0000x reference)
import functools

import jax
import jax.numpy as jnp
from jax import lax
from jax.experimental import pallas as pl
from jax.experimental.pallas import tpu as pltpu
from jax.experimental.pallas import tpu_sc as plsc

B, T, D, V = 16384, 12, 24, 29
NW = 32
ROWS_W = B // NW
DT = D // 8
BT_W = ROWS_W // 128
NSL = ROWS_W // 16


def _ebd_body(x_hbm, lut_hbm, out_hbm, xv, lut_v, buf2, fsem):
    cid = lax.axis_index("c")
    sid = lax.axis_index("s")
    wid = sid * 2 + cid
    rbase = wid * ROWS_W

    pltpu.sync_copy(x_hbm.at[:, pl.ds(rbase, ROWS_W)], xv)
    pltpu.sync_copy(lut_hbm, lut_v)

    def out_at(t):
        return out_hbm.at[t, :, pl.ds(wid * BT_W, BT_W)]

    def t_body(t, carry):
        par = t % 2

        @pl.when(t >= 2)
        def _():
            pltpu.make_async_copy(buf2.at[par], out_at(t), fsem).wait()

        @plsc.parallel_loop(0, NSL, unroll=2)
        def sl_body(i):
            xvec = xv[t, pl.ds(i * 16, 16)]
            xbase = xvec + t * (D * V)
            bt = i // 8
            lo = (i % 8) * 16
            for d in range(D):
                vals = plsc.load_gather(lut_v, [xbase + d * V])
                buf2[par, (d // 8), bt, (d % 8), pl.ds(lo, 16)] = vals

        pltpu.async_copy(buf2.at[par], out_at(t), fsem)
        return carry

    lax.fori_loop(0, T, t_body, 0)
    pltpu.make_async_copy(buf2.at[0], out_at(0), fsem).wait()
    pltpu.make_async_copy(buf2.at[1], out_at(1), fsem).wait()


@jax.jit
def _ebd_gather(xi, lut):
    mesh = plsc.VectorSubcoreMesh(core_axis_name="c", subcore_axis_name="s")
    run = functools.partial(
        pl.kernel,
        out_type=jax.ShapeDtypeStruct((T, DT, B // 128, 8, 128), jnp.float32),
        mesh=mesh,
        scratch_types=[
            pltpu.VMEM((T, ROWS_W), jnp.int32),
            pltpu.VMEM((T * D * V,), jnp.float32),
            pltpu.VMEM((2, DT, BT_W, 8, 128), jnp.float32),
            pltpu.SemaphoreType.DMA,
        ],
        compiler_params=pltpu.CompilerParams(
            use_tc_tiling_on_sc=False, needs_layout_passes=False,
            disable_bounds_checks=True, disable_semaphore_checks=True,
            skip_device_barrier=True),
    )(_ebd_body)
    return run(xi, lut)


def kernel(x, word_ebd, pos_ebd):
    lut = word_ebd.T[None, :, :] + pos_ebd[:, :, None]
    out5 = _ebd_gather(x.T.astype(jnp.int32), lut.reshape(T * D * V))
    return out5.transpose(2, 4, 0, 1, 3).reshape(B, T, D)


# speedup vs baseline: 1.0153x; 1.0153x over previous
import functools

import jax
import jax.numpy as jnp
from jax import lax
from jax.experimental import pallas as pl
from jax.experimental.pallas import tpu as pltpu
from jax.experimental.pallas import tpu_sc as plsc

B, T, D, V = 16384, 12, 24, 29
NW = 32
ROWS_W = B // NW
DT = D // 8
BT_W = ROWS_W // 128
NSL = ROWS_W // 16


def _ebd_body(x_hbm, lut_hbm, out_hbm, xv, lut_v, buf2, fsem):
    cid = lax.axis_index("c")
    sid = lax.axis_index("s")
    wid = sid * 2 + cid
    rbase = wid * ROWS_W

    pltpu.sync_copy(x_hbm.at[:, pl.ds(rbase, ROWS_W)], xv)
    pltpu.sync_copy(lut_hbm, lut_v)

    def out_at(t):
        return out_hbm.at[t, :, pl.ds(wid * BT_W, BT_W)]

    def t_body(t, carry):
        par = t % 2

        @pl.when(t >= 2)
        def _():
            pltpu.make_async_copy(buf2.at[par], out_at(t), fsem).wait()

        @plsc.parallel_loop(0, NSL, unroll=1)
        def sl_body(i):
            xvec = xv[t, pl.ds(i * 16, 16)]
            xbase = xvec + t * (D * V)
            bt = i // 8
            lo = (i % 8) * 16
            for d in range(D):
                vals = plsc.load_gather(lut_v, [xbase + d * V])
                buf2[par, (d // 8), bt, (d % 8), pl.ds(lo, 16)] = vals

        pltpu.async_copy(buf2.at[par], out_at(t), fsem)
        return carry

    lax.fori_loop(0, T, t_body, 0)
    pltpu.make_async_copy(buf2.at[0], out_at(0), fsem).wait()
    pltpu.make_async_copy(buf2.at[1], out_at(1), fsem).wait()


@jax.jit
def _ebd_gather(xi, lut):
    mesh = plsc.VectorSubcoreMesh(core_axis_name="c", subcore_axis_name="s")
    run = functools.partial(
        pl.kernel,
        out_type=jax.ShapeDtypeStruct((T, DT, B // 128, 8, 128), jnp.float32),
        mesh=mesh,
        scratch_types=[
            pltpu.VMEM((T, ROWS_W), jnp.int32),
            pltpu.VMEM((T * D * V,), jnp.float32),
            pltpu.VMEM((2, DT, BT_W, 8, 128), jnp.float32),
            pltpu.SemaphoreType.DMA,
        ],
        compiler_params=pltpu.CompilerParams(
            use_tc_tiling_on_sc=False, needs_layout_passes=False,
            disable_bounds_checks=True, disable_semaphore_checks=True,
            skip_device_barrier=True),
    )(_ebd_body)
    return run(xi, lut)


def kernel(x, word_ebd, pos_ebd):
    lut = word_ebd.T[None, :, :] + pos_ebd[:, :, None]
    out5 = _ebd_gather(x.T.astype(jnp.int32), lut.reshape(T * D * V))
    return out5.transpose(2, 4, 0, 1, 3).reshape(B, T, D)
